# baseline (speedup 1.0000x reference)
import functools

import jax
import jax.numpy as jnp
from jax import lax
from jax.experimental import pallas as pl
from jax.experimental.pallas import tpu as pltpu
from jax.experimental.pallas import tpu_sc as plsc

N_NODES = 10000
N_PAD = 10240
D = 256
HALF = 128
E = 160000
C = 80
NCH = E // C
NS = 16
NC = 2
BN = 1280
EPS = 1e-16


def _h_to_outputs(h, av, ht_ref, aux_ref):
    a_s = jnp.sum(h * av[0:1, :], axis=1)
    a_d = jnp.sum(h * av[1:2, :], axis=1)
    ht_ref[...] = jnp.stack([h[:, :HALF], h[:, HALF:]], axis=0)
    aux_ref[...] = jnp.concatenate(
        [a_s[None], a_d[None], jnp.zeros((6, BN), jnp.float32)], axis=0)


def _tc_first_body(x_ref, w_ref, av_ref, ht_ref, aux_ref):
    h = jnp.dot(x_ref[...], w_ref[...], preferred_element_type=jnp.float32,
                precision=lax.Precision.HIGHEST)
    _h_to_outputs(h, av_ref[...], ht_ref, aux_ref)


def _normalize(acc, den, b):
    x0 = acc[0] / (den[0] + EPS)
    x1 = acc[1] / (den[1] + EPS)
    return jnp.concatenate([x0, x1], axis=1) + b


def _tc_mid_body(accs_ref, den_ref, b_ref, w_ref, av_ref, ht_ref, aux_ref):
    x = jnp.maximum(_normalize(accs_ref[...], den_ref[...], b_ref[...]), 0.0)
    h = jnp.dot(x, w_ref[...], preferred_element_type=jnp.float32,
                precision=lax.Precision.HIGHEST)
    _h_to_outputs(h, av_ref[...], ht_ref, aux_ref)


def _tc_final_body(accs_ref, den_ref, b_ref, out_ref):
    out_ref[...] = _normalize(accs_ref[...], den_ref[...], b_ref[...])


_GRID = (N_PAD // BN,)
_LAYER_OUT_SHAPES = [
    jax.ShapeDtypeStruct((NC, N_PAD, HALF), jnp.float32),
    jax.ShapeDtypeStruct((8, N_PAD), jnp.float32),
]
_LAYER_OUT_SPECS = [
    pl.BlockSpec((NC, BN, HALF), lambda i: (0, i, 0)),
    pl.BlockSpec((8, BN), lambda i: (0, i)),
]
_ACC_SPECS = [
    pl.BlockSpec((NC, BN, HALF), lambda i: (0, i, 0)),
    pl.BlockSpec((NC, BN, 1), lambda i: (0, i, 0)),
    pl.BlockSpec((1, D), lambda i: (0, 0)),
]

_tc_first = pl.pallas_call(
    _tc_first_body,
    grid=_GRID,
    in_specs=[
        pl.BlockSpec((BN, D), lambda i: (i, 0)),
        pl.BlockSpec((D, D), lambda i: (0, 0)),
        pl.BlockSpec((2, D), lambda i: (0, 0)),
    ],
    out_specs=_LAYER_OUT_SPECS,
    out_shape=_LAYER_OUT_SHAPES,
)

_tc_mid = pl.pallas_call(
    _tc_mid_body,
    grid=_GRID,
    in_specs=_ACC_SPECS + [
        pl.BlockSpec((D, D), lambda i: (0, 0)),
        pl.BlockSpec((2, D), lambda i: (0, 0)),
    ],
    out_specs=_LAYER_OUT_SPECS,
    out_shape=_LAYER_OUT_SHAPES,
)

_tc_final = pl.pallas_call(
    _tc_final_body,
    grid=_GRID,
    in_specs=_ACC_SPECS,
    out_specs=pl.BlockSpec((BN, D), lambda i: (i, 0)),
    out_shape=jax.ShapeDtypeStruct((N_PAD, D), jnp.float32),
)


def _sc_edge_body(ht, aux, ei2, accs_out, den_out,
                  acc_s, den_s, as_t, ad_t,
                  sd_t0, sd_t1, dst_t0, dst_t1, idx_t0, idx_t1,
                  p_t0, p_t1, p_sc0, p_sc1, den_t, rows_t0, rows_t1,
                  sd_sem0, sd_sem1, g_sem0, g_sem1,
                  s_sem0, s_sem1, d_sem0, d_sem1):
    c = lax.axis_index("c")
    s = lax.axis_index("s")
    coff = c * N_PAD
    sd_t = (sd_t0, sd_t1)
    dst_t = (dst_t0, dst_t1)
    idx_t = (idx_t0, idx_t1)
    p_t = (p_t0, p_t1)
    p_sc = (p_sc0, p_sc1)
    rows_t = (rows_t0, rows_t1)
    sd_sem = (sd_sem0, sd_sem1)
    g_sem = (g_sem0, g_sem1)
    s_sem = (s_sem0, s_sem1)
    d_sem = (d_sem0, d_sem1)

    pltpu.sync_copy(aux.at[0], as_t)
    pltpu.sync_copy(aux.at[1], ad_t)

    z16 = jnp.zeros((16,), jnp.float32)

    def _zrow(i, carry):
        for j in range(HALF // 16):
            rows_t0[i, pl.ds(j * 16, 16)] = z16
        return carry

    lax.fori_loop(0, C, _zrow, 0)

    def _zden(i, carry):
        den_t[pl.ds(i * 16, 16)] = z16
        return carry

    zrows = N_PAD // NS
    lax.fori_loop(0, zrows // 16, _zden, 0)
    r0 = s * zrows
    for z in range(zrows // C):
        pltpu.sync_copy(rows_t0, acc_s.at[pl.ds(r0 + z * C, C)])
    pltpu.sync_copy(den_t, den_s.at[pl.ds(r0, zrows)])
    plsc.subcore_barrier()

    lane = lax.iota(jnp.int32, 16)
    col0 = jnp.zeros((16,), jnp.int32)
    col1 = col0 + 1
    M = NCH // NS

    def _base(k):
        return (s + k * NS) * C

    def _issue_sd(k, b):
        pltpu.async_copy(ei2.at[pl.ds(_base(k), C)], sd_t[b], sd_sem[b])

    def _wait_sd(b):
        pltpu.make_async_copy(ei2.at[pl.ds(0, C)], sd_t[b], sd_sem[b]).wait()

    def _compute(b):
        for g in range(C // 16):
            rid = lane + (16 * g)
            sv = plsc.load_gather(sd_t[b], [rid, col0])
            dv = plsc.load_gather(sd_t[b], [rid, col1])
            z = plsc.load_gather(as_t, [sv]) + plsc.load_gather(ad_t, [dv])
            z = jnp.maximum(z, 0.2 * z)
            p = jnp.exp(z)
            p_t[b][pl.ds(g * 16, 16)] = p
            p_sc[b][pl.ds(g * 16, 16)] = p
            idx_t[b][pl.ds(g * 16, 16)] = sv + coff
            dst_t[b][pl.ds(g * 16, 16)] = dv

    def _issue_gather(b):
        pltpu.async_copy(ht.at[idx_t[b]], rows_t[b], g_sem[b])

    def _wait_gather(b):
        pltpu.make_async_copy(ht.at[idx_t[b]], rows_t[b], g_sem[b]).wait()

    def _issue_scat(b):
        pltpu.async_copy(rows_t[b], acc_s.at[dst_t[b]], s_sem[b], add=True)
        pltpu.async_copy(p_sc[b], den_s.at[dst_t[b]], d_sem[b], add=True)

    def _wait_scat(b):
        pltpu.make_async_copy(rows_t[b], acc_s.at[dst_t[b]], s_sem[b]).wait()
        pltpu.make_async_copy(p_sc[b], den_s.at[dst_t[b]], d_sem[b]).wait()

    def _scale(b):
        @plsc.parallel_loop(0, C, 1, unroll=8)
        def _srow(e):
            pv = lax.broadcast(p_t[b][pl.ds(e, 16)][0], (16,))
            for j in range(HALF // 16):
                rows_t[b][e, pl.ds(j * 16, 16)] = (
                    rows_t[b][e, pl.ds(j * 16, 16)] * pv)

    def _stage_a(knext, nb, do_sd, do_wscat):
        _wait_sd(nb)
        if do_wscat:
            _wait_scat(nb)
        _compute(nb)
        if do_sd:
            _issue_sd(knext + 2, nb)
        _issue_gather(nb)

    def _stage_b(b):
        _wait_gather(b)
        _scale(b)
        _issue_scat(b)

    _issue_sd(0, 0)
    _issue_sd(1, 1)
    _wait_sd(0)
    _compute(0)
    _issue_sd(2, 0)
    _issue_gather(0)
    _stage_a(1, 1, do_sd=True, do_wscat=False)
    _stage_b(0)

    def _pair(i2, carry):
        k = 1 + 2 * i2
        _stage_a(k + 1, 0, do_sd=True, do_wscat=True)
        _stage_b(1)
        _stage_a(k + 2, 1, do_sd=True, do_wscat=True)
        _stage_b(0)
        return carry

    lax.fori_loop(0, (M - 5) // 2, _pair, 0)
    _stage_a(122, 0, do_sd=True, do_wscat=True)
    _stage_b(1)
    _stage_a(123, 1, do_sd=False, do_wscat=True)
    _stage_b(0)
    _stage_a(124, 0, do_sd=False, do_wscat=True)
    _stage_b(1)
    _stage_b(0)
    _wait_scat(1)
    _wait_scat(0)
    plsc.subcore_barrier()

    pltpu.sync_copy(acc_s.at[pl.ds(r0, zrows)], accs_out.at[c, pl.ds(r0, zrows)])
    pltpu.sync_copy(den_s.at[pl.ds(r0, zrows)], den_out.at[c, pl.ds(r0, zrows)])


_sc_edge = pl.kernel(
    _sc_edge_body,
    out_type=[
        jax.ShapeDtypeStruct((NC, N_PAD, HALF), jnp.float32),
        jax.ShapeDtypeStruct((NC, N_PAD), jnp.float32),
    ],
    mesh=plsc.VectorSubcoreMesh(core_axis_name="c", subcore_axis_name="s"),
    compiler_params=pltpu.CompilerParams(
        needs_layout_passes=False, use_tc_tiling_on_sc=False),
    scratch_types=(
        [
            pltpu.MemorySpace.VMEM_SHARED((N_PAD, HALF), jnp.float32),
            pltpu.MemorySpace.VMEM_SHARED((N_PAD,), jnp.float32),
            pltpu.VMEM((N_PAD,), jnp.float32),
            pltpu.VMEM((N_PAD,), jnp.float32),
        ]
        + 2 * [pltpu.VMEM((C, 2), jnp.int32)]
        + 2 * [pltpu.VMEM((C,), jnp.int32)]
        + 2 * [pltpu.VMEM((C,), jnp.int32)]
        + 2 * [pltpu.VMEM((C + 16,), jnp.float32)]
        + 2 * [pltpu.VMEM((C,), jnp.float32)]
        + [pltpu.VMEM((N_PAD // NS,), jnp.float32)]
        + 2 * [pltpu.VMEM((C, HALF), jnp.float32)]
        + 8 * [pltpu.SemaphoreType.DMA]
    ),
)


def kernel(x, edge_index, W1, a_src1, a_dst1, b1, W2, a_src2, a_dst2, b2):
    x = jnp.asarray(x, jnp.float32)
    ei2 = jnp.asarray(edge_index, jnp.int32).T
    xp = jnp.pad(x, ((0, N_PAD - x.shape[0]), (0, 0)))
    av1 = jnp.stack([a_src1, a_dst1])
    av2 = jnp.stack([a_src2, a_dst2])
    b1r = b1.reshape(1, D)
    b2r = b2.reshape(1, D)

    ht1, aux1 = _tc_first(xp, W1, av1)
    accs1, den1 = _sc_edge(ht1.reshape(NC * N_PAD, HALF), aux1, ei2)
    ht2, aux2 = _tc_mid(accs1, den1.reshape(NC, N_PAD, 1), b1r, W2, av2)
    accs2, den2 = _sc_edge(ht2.reshape(NC * N_PAD, HALF), aux2, ei2)
    out = _tc_final(accs2, den2.reshape(NC, N_PAD, 1), b2r)
    return out[:N_NODES]


# speedup vs baseline: 1.0029x; 1.0029x over previous
import functools

import jax
import jax.numpy as jnp
from jax import lax
from jax.experimental import pallas as pl
from jax.experimental.pallas import tpu as pltpu
from jax.experimental.pallas import tpu_sc as plsc

N_NODES = 10000
N_PAD = 10240
D = 256
HALF = 128
E = 160000
C = 80
NCH = E // C
NS = 16
NC = 2
BN = 1280
EPS = 1e-16


def _h_to_outputs(h, av, ht_ref, aux_ref):
    a_s = jnp.sum(h * av[0:1, :], axis=1)
    a_d = jnp.sum(h * av[1:2, :], axis=1)
    ht_ref[...] = jnp.stack([h[:, :HALF], h[:, HALF:]], axis=0)
    aux_ref[...] = jnp.concatenate(
        [a_s[None], a_d[None], jnp.zeros((6, BN), jnp.float32)], axis=0)


def _tc_first_body(x_ref, w_ref, av_ref, ht_ref, aux_ref):
    h = jnp.dot(x_ref[...], w_ref[...], preferred_element_type=jnp.float32,
                precision=lax.Precision.HIGHEST)
    _h_to_outputs(h, av_ref[...], ht_ref, aux_ref)


def _normalize(acc, den, b):
    x0 = acc[0] / (den[0] + EPS)
    x1 = acc[1] / (den[1] + EPS)
    return jnp.concatenate([x0, x1], axis=1) + b


def _tc_mid_body(accs_ref, den_ref, b_ref, w_ref, av_ref, ht_ref, aux_ref):
    x = jnp.maximum(_normalize(accs_ref[...], den_ref[...], b_ref[...]), 0.0)
    h = jnp.dot(x, w_ref[...], preferred_element_type=jnp.float32,
                precision=lax.Precision.HIGHEST)
    _h_to_outputs(h, av_ref[...], ht_ref, aux_ref)


def _tc_final_body(accs_ref, den_ref, b_ref, out_ref):
    out_ref[...] = _normalize(accs_ref[...], den_ref[...], b_ref[...])


_GRID = (N_PAD // BN,)
_LAYER_OUT_SHAPES = [
    jax.ShapeDtypeStruct((NC, N_PAD, HALF), jnp.float32),
    jax.ShapeDtypeStruct((8, N_PAD), jnp.float32),
]
_LAYER_OUT_SPECS = [
    pl.BlockSpec((NC, BN, HALF), lambda i: (0, i, 0)),
    pl.BlockSpec((8, BN), lambda i: (0, i)),
]
_ACC_SPECS = [
    pl.BlockSpec((NC, BN, HALF), lambda i: (0, i, 0)),
    pl.BlockSpec((NC, BN, 1), lambda i: (0, i, 0)),
    pl.BlockSpec((1, D), lambda i: (0, 0)),
]

_tc_first = pl.pallas_call(
    _tc_first_body,
    grid=_GRID,
    in_specs=[
        pl.BlockSpec((BN, D), lambda i: (i, 0)),
        pl.BlockSpec((D, D), lambda i: (0, 0)),
        pl.BlockSpec((2, D), lambda i: (0, 0)),
    ],
    out_specs=_LAYER_OUT_SPECS,
    out_shape=_LAYER_OUT_SHAPES,
)

_tc_mid = pl.pallas_call(
    _tc_mid_body,
    grid=_GRID,
    in_specs=_ACC_SPECS + [
        pl.BlockSpec((D, D), lambda i: (0, 0)),
        pl.BlockSpec((2, D), lambda i: (0, 0)),
    ],
    out_specs=_LAYER_OUT_SPECS,
    out_shape=_LAYER_OUT_SHAPES,
)

_tc_final = pl.pallas_call(
    _tc_final_body,
    grid=_GRID,
    in_specs=_ACC_SPECS,
    out_specs=pl.BlockSpec((BN, D), lambda i: (i, 0)),
    out_shape=jax.ShapeDtypeStruct((N_PAD, D), jnp.float32),
)


def _sc_edge_body(ht, aux, ei2, accs_out, den_out,
                  acc_s, den_s, as_t, ad_t,
                  sd_t0, sd_t1, dst_t0, dst_t1, idx_t0, idx_t1,
                  p_t0, p_t1, p_sc0, p_sc1, den_t, rows_t0, rows_t1,
                  sd_sem0, sd_sem1, g_sem0, g_sem1,
                  s_sem0, s_sem1, d_sem0, d_sem1):
    c = lax.axis_index("c")
    s = lax.axis_index("s")
    coff = c * N_PAD
    sd_t = (sd_t0, sd_t1)
    dst_t = (dst_t0, dst_t1)
    idx_t = (idx_t0, idx_t1)
    p_t = (p_t0, p_t1)
    p_sc = (p_sc0, p_sc1)
    rows_t = (rows_t0, rows_t1)
    sd_sem = (sd_sem0, sd_sem1)
    g_sem = (g_sem0, g_sem1)
    s_sem = (s_sem0, s_sem1)
    d_sem = (d_sem0, d_sem1)

    pltpu.sync_copy(aux.at[0], as_t)
    pltpu.sync_copy(aux.at[1], ad_t)

    z16 = jnp.zeros((16,), jnp.float32)

    def _zrow(i, carry):
        for j in range(HALF // 16):
            rows_t0[i, pl.ds(j * 16, 16)] = z16
        return carry

    lax.fori_loop(0, C, _zrow, 0)

    def _zden(i, carry):
        den_t[pl.ds(i * 16, 16)] = z16
        return carry

    zrows = N_PAD // NS
    lax.fori_loop(0, zrows // 16, _zden, 0)
    r0 = s * zrows
    for z in range(zrows // C):
        pltpu.sync_copy(rows_t0, acc_s.at[pl.ds(r0 + z * C, C)])
    pltpu.sync_copy(den_t, den_s.at[pl.ds(r0, zrows)])
    plsc.subcore_barrier()

    lane = lax.iota(jnp.int32, 16)
    col0 = jnp.zeros((16,), jnp.int32)
    col1 = col0 + 1
    M = NCH // NS

    def _base(k):
        return (s + k * NS) * C

    def _issue_sd(k, b):
        pltpu.async_copy(ei2.at[pl.ds(_base(k), C)], sd_t[b], sd_sem[b])

    def _wait_sd(b):
        pltpu.make_async_copy(ei2.at[pl.ds(0, C)], sd_t[b], sd_sem[b]).wait()

    def _compute(b):
        for g in range(C // 16):
            rid = lane + (16 * g)
            sv = plsc.load_gather(sd_t[b], [rid, col0])
            dv = plsc.load_gather(sd_t[b], [rid, col1])
            z = plsc.load_gather(as_t, [sv]) + plsc.load_gather(ad_t, [dv])
            z = jnp.maximum(z, 0.2 * z)
            p = jnp.exp(z)
            p_t[b][pl.ds(g * 16, 16)] = p
            p_sc[b][pl.ds(g * 16, 16)] = p
            idx_t[b][pl.ds(g * 16, 16)] = sv + coff
            dst_t[b][pl.ds(g * 16, 16)] = dv

    def _issue_gather(b):
        pltpu.async_copy(ht.at[idx_t[b]], rows_t[b], g_sem[b])

    def _wait_gather(b):
        pltpu.make_async_copy(ht.at[idx_t[b]], rows_t[b], g_sem[b]).wait()

    def _issue_scat(b):
        pltpu.async_copy(rows_t[b], acc_s.at[dst_t[b]], s_sem[b], add=True)
        pltpu.async_copy(p_sc[b], den_s.at[dst_t[b]], d_sem[b], add=True)

    def _wait_scat(b):
        pltpu.make_async_copy(rows_t[b], acc_s.at[dst_t[b]], s_sem[b]).wait()
        pltpu.make_async_copy(p_sc[b], den_s.at[dst_t[b]], d_sem[b]).wait()

    def _scale(b):
        @plsc.parallel_loop(0, C, 1, unroll=4)
        def _srow(e):
            pv = lax.broadcast(p_t[b][pl.ds(e, 16)][0], (16,))
            for j in range(HALF // 16):
                rows_t[b][e, pl.ds(j * 16, 16)] = (
                    rows_t[b][e, pl.ds(j * 16, 16)] * pv)

    def _stage_a(knext, nb, do_sd, do_wscat):
        _wait_sd(nb)
        if do_wscat:
            _wait_scat(nb)
        _compute(nb)
        if do_sd:
            _issue_sd(knext + 2, nb)
        _issue_gather(nb)

    def _stage_b(b):
        _wait_gather(b)
        _scale(b)
        _issue_scat(b)

    _issue_sd(0, 0)
    _issue_sd(1, 1)
    _wait_sd(0)
    _compute(0)
    _issue_sd(2, 0)
    _issue_gather(0)
    _stage_a(1, 1, do_sd=True, do_wscat=False)
    _stage_b(0)

    def _pair(i2, carry):
        k = 1 + 2 * i2
        _stage_a(k + 1, 0, do_sd=True, do_wscat=True)
        _stage_b(1)
        _stage_a(k + 2, 1, do_sd=True, do_wscat=True)
        _stage_b(0)
        return carry

    lax.fori_loop(0, (M - 5) // 2, _pair, 0)
    _stage_a(122, 0, do_sd=True, do_wscat=True)
    _stage_b(1)
    _stage_a(123, 1, do_sd=False, do_wscat=True)
    _stage_b(0)
    _stage_a(124, 0, do_sd=False, do_wscat=True)
    _stage_b(1)
    _stage_b(0)
    _wait_scat(1)
    _wait_scat(0)
    plsc.subcore_barrier()

    pltpu.sync_copy(acc_s.at[pl.ds(r0, zrows)], accs_out.at[c, pl.ds(r0, zrows)])
    pltpu.sync_copy(den_s.at[pl.ds(r0, zrows)], den_out.at[c, pl.ds(r0, zrows)])


_sc_edge = pl.kernel(
    _sc_edge_body,
    out_type=[
        jax.ShapeDtypeStruct((NC, N_PAD, HALF), jnp.float32),
        jax.ShapeDtypeStruct((NC, N_PAD), jnp.float32),
    ],
    mesh=plsc.VectorSubcoreMesh(core_axis_name="c", subcore_axis_name="s"),
    compiler_params=pltpu.CompilerParams(
        needs_layout_passes=False, use_tc_tiling_on_sc=False),
    scratch_types=(
        [
            pltpu.MemorySpace.VMEM_SHARED((N_PAD, HALF), jnp.float32),
            pltpu.MemorySpace.VMEM_SHARED((N_PAD,), jnp.float32),
            pltpu.VMEM((N_PAD,), jnp.float32),
            pltpu.VMEM((N_PAD,), jnp.float32),
        ]
        + 2 * [pltpu.VMEM((C, 2), jnp.int32)]
        + 2 * [pltpu.VMEM((C,), jnp.int32)]
        + 2 * [pltpu.VMEM((C,), jnp.int32)]
        + 2 * [pltpu.VMEM((C + 16,), jnp.float32)]
        + 2 * [pltpu.VMEM((C,), jnp.float32)]
        + [pltpu.VMEM((N_PAD // NS,), jnp.float32)]
        + 2 * [pltpu.VMEM((C, HALF), jnp.float32)]
        + 8 * [pltpu.SemaphoreType.DMA]
    ),
)


def kernel(x, edge_index, W1, a_src1, a_dst1, b1, W2, a_src2, a_dst2, b2):
    x = jnp.asarray(x, jnp.float32)
    ei2 = jnp.asarray(edge_index, jnp.int32).T
    xp = jnp.pad(x, ((0, N_PAD - x.shape[0]), (0, 0)))
    av1 = jnp.stack([a_src1, a_dst1])
    av2 = jnp.stack([a_src2, a_dst2])
    b1r = b1.reshape(1, D)
    b2r = b2.reshape(1, D)

    ht1, aux1 = _tc_first(xp, W1, av1)
    accs1, den1 = _sc_edge(ht1.reshape(NC * N_PAD, HALF), aux1, ei2)
    ht2, aux2 = _tc_mid(accs1, den1.reshape(NC, N_PAD, 1), b1r, W2, av2)
    accs2, den2 = _sc_edge(ht2.reshape(NC * N_PAD, HALF), aux2, ei2)
    out = _tc_final(accs2, den2.reshape(NC, N_PAD, 1), b2r)
    return out[:N_NODES]


# speedup vs baseline: 1.0188x; 1.0159x over previous
import functools

import jax
import jax.numpy as jnp
from jax import lax
from jax.experimental import pallas as pl
from jax.experimental.pallas import tpu as pltpu
from jax.experimental.pallas import tpu_sc as plsc

N_NODES = 10000
N_PAD = 10240
D = 256
HALF = 128
E = 160000
C = 80
NCH = E // C
NS = 16
NC = 2
BN = 1280
EPS = 1e-16


def _h_to_outputs(h, av, ht_ref, aux_ref):
    a_s = jnp.sum(h * av[0:1, :], axis=1)
    a_d = jnp.sum(h * av[1:2, :], axis=1)
    ht_ref[...] = jnp.stack([h[:, :HALF], h[:, HALF:]], axis=0)
    aux_ref[...] = jnp.concatenate(
        [a_s[None], a_d[None], jnp.zeros((6, BN), jnp.float32)], axis=0)


def _tc_first_body(x_ref, w_ref, av_ref, ht_ref, aux_ref):
    h = jnp.dot(x_ref[...], w_ref[...], preferred_element_type=jnp.float32,
                precision=lax.Precision.HIGHEST)
    _h_to_outputs(h, av_ref[...], ht_ref, aux_ref)


def _normalize(acc, den, b):
    x0 = acc[0] / (den[0] + EPS)
    x1 = acc[1] / (den[1] + EPS)
    return jnp.concatenate([x0, x1], axis=1) + b


def _tc_mid_body(accs_ref, den_ref, b_ref, w_ref, av_ref, ht_ref, aux_ref):
    x = jnp.maximum(_normalize(accs_ref[...], den_ref[...], b_ref[...]), 0.0)
    h = jnp.dot(x, w_ref[...], preferred_element_type=jnp.float32,
                precision=lax.Precision.HIGHEST)
    _h_to_outputs(h, av_ref[...], ht_ref, aux_ref)


def _tc_final_body(accs_ref, den_ref, b_ref, out_ref):
    out_ref[...] = _normalize(accs_ref[...], den_ref[...], b_ref[...])


_GRID = (N_PAD // BN,)
_LAYER_OUT_SHAPES = [
    jax.ShapeDtypeStruct((NC, N_PAD, HALF), jnp.float32),
    jax.ShapeDtypeStruct((8, N_PAD), jnp.float32),
]
_LAYER_OUT_SPECS = [
    pl.BlockSpec((NC, BN, HALF), lambda i: (0, i, 0)),
    pl.BlockSpec((8, BN), lambda i: (0, i)),
]
_ACC_SPECS = [
    pl.BlockSpec((NC, BN, HALF), lambda i: (0, i, 0)),
    pl.BlockSpec((NC, BN, 1), lambda i: (0, i, 0)),
    pl.BlockSpec((1, D), lambda i: (0, 0)),
]

_tc_first = pl.pallas_call(
    _tc_first_body,
    grid=_GRID,
    in_specs=[
        pl.BlockSpec((BN, D), lambda i: (i, 0)),
        pl.BlockSpec((D, D), lambda i: (0, 0)),
        pl.BlockSpec((2, D), lambda i: (0, 0)),
    ],
    out_specs=_LAYER_OUT_SPECS,
    out_shape=_LAYER_OUT_SHAPES,
)

_tc_mid = pl.pallas_call(
    _tc_mid_body,
    grid=_GRID,
    in_specs=_ACC_SPECS + [
        pl.BlockSpec((D, D), lambda i: (0, 0)),
        pl.BlockSpec((2, D), lambda i: (0, 0)),
    ],
    out_specs=_LAYER_OUT_SPECS,
    out_shape=_LAYER_OUT_SHAPES,
)

BF = 2000

_tc_final = pl.pallas_call(
    _tc_final_body,
    grid=(N_NODES // BF,),
    in_specs=[
        pl.BlockSpec((NC, BF, HALF), lambda i: (0, i, 0)),
        pl.BlockSpec((NC, BF, 1), lambda i: (0, i, 0)),
        pl.BlockSpec((1, D), lambda i: (0, 0)),
    ],
    out_specs=pl.BlockSpec((BF, D), lambda i: (i, 0)),
    out_shape=jax.ShapeDtypeStruct((N_NODES, D), jnp.float32),
)


def _sc_edge_body(ht, aux, ei2, accs_out, den_out,
                  acc_s, den_s, as_t, ad_t,
                  sd_t0, sd_t1, dst_t0, dst_t1, idx_t0, idx_t1,
                  p_t0, p_t1, p_sc0, p_sc1, den_t, rows_t0, rows_t1,
                  sd_sem0, sd_sem1, g_sem0, g_sem1,
                  s_sem0, s_sem1, d_sem0, d_sem1):
    c = lax.axis_index("c")
    s = lax.axis_index("s")
    coff = c * N_PAD
    sd_t = (sd_t0, sd_t1)
    dst_t = (dst_t0, dst_t1)
    idx_t = (idx_t0, idx_t1)
    p_t = (p_t0, p_t1)
    p_sc = (p_sc0, p_sc1)
    rows_t = (rows_t0, rows_t1)
    sd_sem = (sd_sem0, sd_sem1)
    g_sem = (g_sem0, g_sem1)
    s_sem = (s_sem0, s_sem1)
    d_sem = (d_sem0, d_sem1)

    pltpu.sync_copy(aux.at[0], as_t)
    pltpu.sync_copy(aux.at[1], ad_t)

    z16 = jnp.zeros((16,), jnp.float32)

    def _zrow(i, carry):
        for j in range(HALF // 16):
            rows_t0[i, pl.ds(j * 16, 16)] = z16
        return carry

    lax.fori_loop(0, C, _zrow, 0)

    def _zden(i, carry):
        den_t[pl.ds(i * 16, 16)] = z16
        return carry

    zrows = N_PAD // NS
    lax.fori_loop(0, zrows // 16, _zden, 0)
    r0 = s * zrows
    for z in range(zrows // C):
        pltpu.sync_copy(rows_t0, acc_s.at[pl.ds(r0 + z * C, C)])
    pltpu.sync_copy(den_t, den_s.at[pl.ds(r0, zrows)])
    plsc.subcore_barrier()

    lane = lax.iota(jnp.int32, 16)
    col0 = jnp.zeros((16,), jnp.int32)
    col1 = col0 + 1
    M = NCH // NS

    def _base(k):
        return (s + k * NS) * C

    def _issue_sd(k, b):
        pltpu.async_copy(ei2.at[pl.ds(_base(k), C)], sd_t[b], sd_sem[b])

    def _wait_sd(b):
        pltpu.make_async_copy(ei2.at[pl.ds(0, C)], sd_t[b], sd_sem[b]).wait()

    def _compute(b):
        for g in range(C // 16):
            rid = lane + (16 * g)
            sv = plsc.load_gather(sd_t[b], [rid, col0])
            dv = plsc.load_gather(sd_t[b], [rid, col1])
            z = plsc.load_gather(as_t, [sv]) + plsc.load_gather(ad_t, [dv])
            z = jnp.maximum(z, 0.2 * z)
            p = jnp.exp(z)
            p_t[b][pl.ds(g * 16, 16)] = p
            p_sc[b][pl.ds(g * 16, 16)] = p
            idx_t[b][pl.ds(g * 16, 16)] = sv + coff
            dst_t[b][pl.ds(g * 16, 16)] = dv

    def _issue_gather(b):
        pltpu.async_copy(ht.at[idx_t[b]], rows_t[b], g_sem[b])

    def _wait_gather(b):
        pltpu.make_async_copy(ht.at[idx_t[b]], rows_t[b], g_sem[b]).wait()

    def _issue_scat(b):
        pltpu.async_copy(rows_t[b], acc_s.at[dst_t[b]], s_sem[b], add=True)
        pltpu.async_copy(p_sc[b], den_s.at[dst_t[b]], d_sem[b], add=True)

    def _wait_scat(b):
        pltpu.make_async_copy(rows_t[b], acc_s.at[dst_t[b]], s_sem[b]).wait()
        pltpu.make_async_copy(p_sc[b], den_s.at[dst_t[b]], d_sem[b]).wait()

    def _scale(b):
        @plsc.parallel_loop(0, C, 1, unroll=4)
        def _srow(e):
            pv = lax.broadcast(p_t[b][pl.ds(e, 16)][0], (16,))
            for j in range(HALF // 16):
                rows_t[b][e, pl.ds(j * 16, 16)] = (
                    rows_t[b][e, pl.ds(j * 16, 16)] * pv)

    def _stage_a(knext, nb, do_sd, do_wscat):
        _wait_sd(nb)
        if do_wscat:
            _wait_scat(nb)
        _compute(nb)
        if do_sd:
            _issue_sd(knext + 2, nb)
        _issue_gather(nb)

    def _stage_b(b):
        _wait_gather(b)
        _scale(b)
        _issue_scat(b)

    _issue_sd(0, 0)
    _issue_sd(1, 1)
    _wait_sd(0)
    _compute(0)
    _issue_sd(2, 0)
    _issue_gather(0)
    _stage_a(1, 1, do_sd=True, do_wscat=False)
    _stage_b(0)

    def _pair(i2, carry):
        k = 1 + 2 * i2
        _stage_a(k + 1, 0, do_sd=True, do_wscat=True)
        _stage_b(1)
        _stage_a(k + 2, 1, do_sd=True, do_wscat=True)
        _stage_b(0)
        return carry

    lax.fori_loop(0, (M - 5) // 2, _pair, 0)
    _stage_a(122, 0, do_sd=True, do_wscat=True)
    _stage_b(1)
    _stage_a(123, 1, do_sd=False, do_wscat=True)
    _stage_b(0)
    _stage_a(124, 0, do_sd=False, do_wscat=True)
    _stage_b(1)
    _stage_b(0)
    _wait_scat(1)
    _wait_scat(0)
    plsc.subcore_barrier()

    pltpu.sync_copy(acc_s.at[pl.ds(r0, zrows)], accs_out.at[c, pl.ds(r0, zrows)])
    pltpu.sync_copy(den_s.at[pl.ds(r0, zrows)], den_out.at[c, pl.ds(r0, zrows)])


_sc_edge = pl.kernel(
    _sc_edge_body,
    out_type=[
        jax.ShapeDtypeStruct((NC, N_PAD, HALF), jnp.float32),
        jax.ShapeDtypeStruct((NC, N_PAD), jnp.float32),
    ],
    mesh=plsc.VectorSubcoreMesh(core_axis_name="c", subcore_axis_name="s"),
    compiler_params=pltpu.CompilerParams(
        needs_layout_passes=False, use_tc_tiling_on_sc=False),
    scratch_types=(
        [
            pltpu.MemorySpace.VMEM_SHARED((N_PAD, HALF), jnp.float32),
            pltpu.MemorySpace.VMEM_SHARED((N_PAD,), jnp.float32),
            pltpu.VMEM((N_PAD,), jnp.float32),
            pltpu.VMEM((N_PAD,), jnp.float32),
        ]
        + 2 * [pltpu.VMEM((C, 2), jnp.int32)]
        + 2 * [pltpu.VMEM((C,), jnp.int32)]
        + 2 * [pltpu.VMEM((C,), jnp.int32)]
        + 2 * [pltpu.VMEM((C + 16,), jnp.float32)]
        + 2 * [pltpu.VMEM((C,), jnp.float32)]
        + [pltpu.VMEM((N_PAD // NS,), jnp.float32)]
        + 2 * [pltpu.VMEM((C, HALF), jnp.float32)]
        + 8 * [pltpu.SemaphoreType.DMA]
    ),
)


def kernel(x, edge_index, W1, a_src1, a_dst1, b1, W2, a_src2, a_dst2, b2):
    x = jnp.asarray(x, jnp.float32)
    ei2 = jnp.asarray(edge_index, jnp.int32).T
    xp = jnp.pad(x, ((0, N_PAD - x.shape[0]), (0, 0)))
    av1 = jnp.stack([a_src1, a_dst1])
    av2 = jnp.stack([a_src2, a_dst2])
    b1r = b1.reshape(1, D)
    b2r = b2.reshape(1, D)

    ht1, aux1 = _tc_first(xp, W1, av1)
    accs1, den1 = _sc_edge(ht1.reshape(NC * N_PAD, HALF), aux1, ei2)
    ht2, aux2 = _tc_mid(accs1, den1.reshape(NC, N_PAD, 1), b1r, W2, av2)
    accs2, den2 = _sc_edge(ht2.reshape(NC * N_PAD, HALF), aux2, ei2)
    return _tc_final(accs2, den2.reshape(NC, N_PAD, 1), b2r)


# speedup vs baseline: 1.3867x; 1.3612x over previous
import functools

import jax
import jax.numpy as jnp
from jax import lax
from jax.experimental import pallas as pl
from jax.experimental.pallas import tpu as pltpu
from jax.experimental.pallas import tpu_sc as plsc

N_NODES = 10000
N_PAD = 10240
D = 256
HALF = 128
E = 160000
C = 80
NCH = E // C
NS = 16
NC = 2
BN = 1280
EPS = 1e-16


def _h_to_outputs(h, av, ht_ref, aux_ref):
    a_s = jnp.sum(h * av[0:1, :], axis=1)
    a_d = jnp.sum(h * av[1:2, :], axis=1)
    ht_ref[...] = jnp.stack([h[:, :HALF], h[:, HALF:]], axis=0)
    aux_ref[...] = jnp.concatenate(
        [a_s[None], a_d[None], jnp.zeros((6, BN), jnp.float32)], axis=0)


def _tc_first_body(x_ref, w_ref, av_ref, ht_ref, aux_ref):
    h = jnp.dot(x_ref[...], w_ref[...], preferred_element_type=jnp.float32,
                precision=lax.Precision.HIGHEST)
    _h_to_outputs(h, av_ref[...], ht_ref, aux_ref)


def _normalize(acc, den, b):
    x0 = acc[0] / (den[0] + EPS)
    x1 = acc[1] / (den[1] + EPS)
    return jnp.concatenate([x0, x1], axis=1) + b


def _tc_mid_body(accs_ref, den_ref, b_ref, w_ref, av_ref, ht_ref, aux_ref):
    x = jnp.maximum(_normalize(accs_ref[...], den_ref[...], b_ref[...]), 0.0)
    h = jnp.dot(x, w_ref[...], preferred_element_type=jnp.float32,
                precision=lax.Precision.HIGHEST)
    _h_to_outputs(h, av_ref[...], ht_ref, aux_ref)


def _tc_final_body(accs_ref, den_ref, b_ref, out_ref):
    out_ref[...] = _normalize(accs_ref[...], den_ref[...], b_ref[...])


_GRID = (N_PAD // BN,)
_LAYER_OUT_SHAPES = [
    jax.ShapeDtypeStruct((NC, N_PAD, HALF), jnp.float32),
    jax.ShapeDtypeStruct((8, N_PAD), jnp.float32),
]
_LAYER_OUT_SPECS = [
    pl.BlockSpec((NC, BN, HALF), lambda i: (0, i, 0)),
    pl.BlockSpec((8, BN), lambda i: (0, i)),
]
_ACC_SPECS = [
    pl.BlockSpec((NC, BN, HALF), lambda i: (0, i, 0)),
    pl.BlockSpec((NC, BN, 1), lambda i: (0, i, 0)),
    pl.BlockSpec((1, D), lambda i: (0, 0)),
]

_tc_first = pl.pallas_call(
    _tc_first_body,
    grid=_GRID,
    in_specs=[
        pl.BlockSpec((BN, D), lambda i: (i, 0)),
        pl.BlockSpec((D, D), lambda i: (0, 0)),
        pl.BlockSpec((2, D), lambda i: (0, 0)),
    ],
    out_specs=_LAYER_OUT_SPECS,
    out_shape=_LAYER_OUT_SHAPES,
)

_tc_mid = pl.pallas_call(
    _tc_mid_body,
    grid=_GRID,
    in_specs=_ACC_SPECS + [
        pl.BlockSpec((D, D), lambda i: (0, 0)),
        pl.BlockSpec((2, D), lambda i: (0, 0)),
    ],
    out_specs=_LAYER_OUT_SPECS,
    out_shape=_LAYER_OUT_SHAPES,
)

BF = 2000

_tc_final = pl.pallas_call(
    _tc_final_body,
    grid=(N_NODES // BF,),
    in_specs=[
        pl.BlockSpec((NC, BF, HALF), lambda i: (0, i, 0)),
        pl.BlockSpec((NC, BF, 1), lambda i: (0, i, 0)),
        pl.BlockSpec((1, D), lambda i: (0, 0)),
    ],
    out_specs=pl.BlockSpec((BF, D), lambda i: (i, 0)),
    out_shape=jax.ShapeDtypeStruct((N_NODES, D), jnp.float32),
)


def _sc_edge_body(ht, aux, ei, accs_out, den_out,
                  acc_s, den_s, as_t, ad_t,
                  src_t0, src_t1, dst_t0, dst_t1, idx_t0, idx_t1,
                  p_t0, p_t1, p_sc0, p_sc1, den_t, rows_t0, rows_t1,
                  sd_sem0, sd_sem1, g_sem0, g_sem1,
                  s_sem0, s_sem1, d_sem0, d_sem1):
    c = lax.axis_index("c")
    s = lax.axis_index("s")
    coff = c * N_PAD
    src_t = (src_t0, src_t1)
    dst_t = (dst_t0, dst_t1)
    idx_t = (idx_t0, idx_t1)
    p_t = (p_t0, p_t1)
    p_sc = (p_sc0, p_sc1)
    rows_t = (rows_t0, rows_t1)
    sd_sem = (sd_sem0, sd_sem1)
    g_sem = (g_sem0, g_sem1)
    s_sem = (s_sem0, s_sem1)
    d_sem = (d_sem0, d_sem1)

    pltpu.sync_copy(aux.at[0], as_t)
    pltpu.sync_copy(aux.at[1], ad_t)

    z16 = jnp.zeros((16,), jnp.float32)

    def _zrow(i, carry):
        for j in range(HALF // 16):
            rows_t0[i, pl.ds(j * 16, 16)] = z16
        return carry

    lax.fori_loop(0, C, _zrow, 0)

    def _zden(i, carry):
        den_t[pl.ds(i * 16, 16)] = z16
        return carry

    zrows = N_PAD // NS
    lax.fori_loop(0, zrows // 16, _zden, 0)
    r0 = s * zrows
    zcopies = [
        pltpu.async_copy(rows_t0, acc_s.at[pl.ds(r0 + z * C, C)], g_sem0)
        for z in range(zrows // C)
    ]
    pltpu.async_copy(den_t, den_s.at[pl.ds(r0, zrows)], g_sem1)
    for zc in zcopies:
        zc.wait()
    pltpu.make_async_copy(den_t, den_s.at[pl.ds(r0, zrows)], g_sem1).wait()
    plsc.subcore_barrier()

    M = NCH // NS

    def _base(k):
        return (s + k * NS) * C

    def _issue_sd(k, b):
        pltpu.async_copy(ei.at[0, pl.ds(_base(k), C)], src_t[b], sd_sem[b])
        pltpu.async_copy(ei.at[1, pl.ds(_base(k), C)], dst_t[b], sd_sem[b])

    def _wait_sd(b):
        pltpu.make_async_copy(ei.at[0, pl.ds(0, C)], src_t[b], sd_sem[b]).wait()
        pltpu.make_async_copy(ei.at[1, pl.ds(0, C)], dst_t[b], sd_sem[b]).wait()

    def _compute(b):
        for g in range(C // 16):
            sv = src_t[b][pl.ds(g * 16, 16)]
            dv = dst_t[b][pl.ds(g * 16, 16)]
            z = plsc.load_gather(as_t, [sv]) + plsc.load_gather(ad_t, [dv])
            z = jnp.maximum(z, 0.2 * z)
            p = jnp.exp(z)
            p_t[b][pl.ds(g * 16, 16)] = p
            p_sc[b][pl.ds(g * 16, 16)] = p
            idx_t[b][pl.ds(g * 16, 16)] = sv + coff

    def _issue_gather(b):
        pltpu.async_copy(ht.at[idx_t[b]], rows_t[b], g_sem[b])

    def _wait_gather(b):
        pltpu.make_async_copy(ht.at[idx_t[b]], rows_t[b], g_sem[b]).wait()

    def _issue_scat(b):
        pltpu.async_copy(rows_t[b], acc_s.at[dst_t[b]], s_sem[b], add=True)
        pltpu.async_copy(p_sc[b], den_s.at[dst_t[b]], d_sem[b], add=True)

    def _wait_scat(b):
        pltpu.make_async_copy(rows_t[b], acc_s.at[dst_t[b]], s_sem[b]).wait()
        pltpu.make_async_copy(p_sc[b], den_s.at[dst_t[b]], d_sem[b]).wait()

    def _scale(b):
        @plsc.parallel_loop(0, C, 1, unroll=4)
        def _srow(e):
            pv = lax.broadcast(p_t[b][pl.ds(e, 16)][0], (16,))
            for j in range(HALF // 16):
                rows_t[b][e, pl.ds(j * 16, 16)] = (
                    rows_t[b][e, pl.ds(j * 16, 16)] * pv)

    def _stage_a(knext, nb, do_sd, do_wscat):
        _wait_sd(nb)
        if do_wscat:
            _wait_scat(nb)
        _compute(nb)
        if do_sd:
            _issue_sd(knext + 2, nb)
        _issue_gather(nb)

    def _stage_b(b):
        _wait_gather(b)
        _scale(b)
        _issue_scat(b)

    _issue_sd(0, 0)
    _issue_sd(1, 1)
    _wait_sd(0)
    _compute(0)
    _issue_sd(2, 0)
    _issue_gather(0)
    _stage_a(1, 1, do_sd=True, do_wscat=False)
    _stage_b(0)

    def _pair(i2, carry):
        k = 1 + 2 * i2
        _stage_a(k + 1, 0, do_sd=True, do_wscat=True)
        _stage_b(1)
        _stage_a(k + 2, 1, do_sd=True, do_wscat=True)
        _stage_b(0)
        return carry

    lax.fori_loop(0, (M - 5) // 2, _pair, 0)
    _stage_a(122, 0, do_sd=True, do_wscat=True)
    _stage_b(1)
    _stage_a(123, 1, do_sd=False, do_wscat=True)
    _stage_b(0)
    _stage_a(124, 0, do_sd=False, do_wscat=True)
    _stage_b(1)
    _stage_b(0)
    _wait_scat(1)
    _wait_scat(0)
    plsc.subcore_barrier()

    pltpu.sync_copy(acc_s.at[pl.ds(r0, zrows)], accs_out.at[c, pl.ds(r0, zrows)])
    pltpu.sync_copy(den_s.at[pl.ds(r0, zrows)], den_out.at[c, pl.ds(r0, zrows)])


_sc_edge = pl.kernel(
    _sc_edge_body,
    out_type=[
        jax.ShapeDtypeStruct((NC, N_PAD, HALF), jnp.float32),
        jax.ShapeDtypeStruct((NC, N_PAD), jnp.float32),
    ],
    mesh=plsc.VectorSubcoreMesh(core_axis_name="c", subcore_axis_name="s"),
    compiler_params=pltpu.CompilerParams(
        needs_layout_passes=False, use_tc_tiling_on_sc=False),
    scratch_types=(
        [
            pltpu.MemorySpace.VMEM_SHARED((N_PAD, HALF), jnp.float32),
            pltpu.MemorySpace.VMEM_SHARED((N_PAD,), jnp.float32),
            pltpu.VMEM((N_PAD,), jnp.float32),
            pltpu.VMEM((N_PAD,), jnp.float32),
        ]
        + 2 * [pltpu.VMEM((C,), jnp.int32)]
        + 2 * [pltpu.VMEM((C,), jnp.int32)]
        + 2 * [pltpu.VMEM((C,), jnp.int32)]
        + 2 * [pltpu.VMEM((C + 16,), jnp.float32)]
        + 2 * [pltpu.VMEM((C,), jnp.float32)]
        + [pltpu.VMEM((N_PAD // NS,), jnp.float32)]
        + 2 * [pltpu.VMEM((C, HALF), jnp.float32)]
        + 8 * [pltpu.SemaphoreType.DMA]
    ),
)


def kernel(x, edge_index, W1, a_src1, a_dst1, b1, W2, a_src2, a_dst2, b2):
    x = jnp.asarray(x, jnp.float32)
    ei = jnp.asarray(edge_index, jnp.int32)
    xp = jnp.pad(x, ((0, N_PAD - x.shape[0]), (0, 0)))
    av1 = jnp.stack([a_src1, a_dst1])
    av2 = jnp.stack([a_src2, a_dst2])
    b1r = b1.reshape(1, D)
    b2r = b2.reshape(1, D)

    ht1, aux1 = _tc_first(xp, W1, av1)
    accs1, den1 = _sc_edge(ht1.reshape(NC * N_PAD, HALF), aux1, ei)
    ht2, aux2 = _tc_mid(accs1, den1.reshape(NC, N_PAD, 1), b1r, W2, av2)
    accs2, den2 = _sc_edge(ht2.reshape(NC * N_PAD, HALF), aux2, ei)
    return _tc_final(accs2, den2.reshape(NC, N_PAD, 1), b2r)
